# baseline (speedup 1.0000x reference)
import functools

import jax
import jax.numpy as jnp
from jax import lax
from jax.experimental import pallas as pl
from jax.experimental.pallas import tpu as pltpu
from jax.experimental.pallas import tpu_sc as plsc

CTX = 200
PATH = 20
EMBED = 64
LANES = 16
NVREG = EMBED // LANES
CTX_ROWS = 2
CTX_PAD = CTX_ROWS * 128
PATH_PAD = 32
W = 2 * EMBED

_mesh = plsc.VectorSubcoreMesh(core_axis_name="c", subcore_axis_name="s")


@functools.partial(
    pl.kernel,
    out_type=(
        jax.ShapeDtypeStruct((EMBED,), jnp.float32),
        jax.ShapeDtypeStruct((PATH_PAD, W), jnp.float32),
    ),
    mesh=_mesh,
    scratch_types=[
        pltpu.VMEM((CTX_ROWS, 128), jnp.int32),
        pltpu.VMEM((PATH_PAD,), jnp.int32),
        pltpu.VMEM((CTX_PAD, LANES), jnp.float32),
        pltpu.VMEM((CTX_PAD, W), jnp.float32),
        pltpu.VMEM((PATH_PAD, W), jnp.float32),
        pltpu.VMEM((EMBED,), jnp.float32),
        pltpu.SemaphoreType.DMA,
    ],
)
def _sc_gather_pool(ctx_idx2_hbm, path_idx2_hbm, parf_hbm, ctx_table_hbm,
                    node_table_hbm, h_hbm, nrows_hbm, idx_v, pidx_v, parf_v,
                    crows_v, nrows_v, h_v, sem):
    wid = lax.axis_index("s") * _mesh.num_cores + lax.axis_index("c")

    @pl.when(wid == 0)
    def _():
        pltpu.sync_copy(ctx_idx2_hbm, idx_v)
        pltpu.sync_copy(path_idx2_hbm, pidx_v)
        pltpu.sync_copy(parf_hbm, parf_v)
        c0 = pltpu.async_copy(ctx_table_hbm.at[idx_v.at[0]],
                              crows_v.at[pl.ds(0, 128)], sem)
        c1 = pltpu.async_copy(ctx_table_hbm.at[idx_v.at[1]],
                              crows_v.at[pl.ds(128, 128)], sem)
        c2 = pltpu.async_copy(node_table_hbm.at[pidx_v], nrows_v, sem)
        c0.wait()
        c1.wait()
        c2.wait()

        def make_body(g):
            def body(i, acc):
                par = parf_v[128 * g + i, pl.ds(0, LANES)]
                out = []
                for k in range(NVREG):
                    lo = crows_v[128 * g + i, pl.ds(LANES * k, LANES)]
                    hi = crows_v[128 * g + i, pl.ds(EMBED + LANES * k, LANES)]
                    out.append(acc[k] + lo + par * (hi - lo))
                return tuple(out)
            return body

        acc = tuple(jnp.zeros((LANES,), jnp.float32) for _ in range(NVREG))
        acc = lax.fori_loop(0, 128, make_body(0), acc)
        acc = lax.fori_loop(0, CTX - 128, make_body(1), acc)
        for k in range(NVREG):
            h_v[pl.ds(LANES * k, LANES)] = acc[k] * (1.0 / CTX)

        pltpu.sync_copy(h_v, h_hbm)
        pltpu.sync_copy(nrows_v, nrows_hbm)


def _loss_body(h_ref, n_ref, par_ref, bits_ref, mask_ref, o_ref):
    h = h_ref[...]
    n = n_ref[...]
    par = par_ref[...]
    b = bits_ref[...]
    m = mask_ref[...]
    lo = n[:, :EMBED]
    hi = n[:, EMBED:]
    nsel = lo + par * (hi - lo)
    t = jnp.sum(nsel * h, axis=1, keepdims=True)
    s = jax.nn.sigmoid(t)
    eps = 1e-9
    per = -b * jnp.log(s + eps) - (1.0 - b) * jnp.log(1.0 - s + eps)
    o_ref[0, 0] = jnp.sum(per * m)


_loss_call = pl.pallas_call(
    _loss_body,
    out_shape=jax.ShapeDtypeStruct((1, 1), jnp.float32),
    out_specs=pl.BlockSpec(memory_space=pltpu.SMEM),
)


def kernel(context_idx, path_indices, code_bits, context_table, node_table):
    ctx = jnp.asarray(context_idx, jnp.int32)
    pidx = jnp.asarray(path_indices, jnp.int32)
    ctx2_pad = (jnp.zeros((CTX_PAD,), jnp.int32)
                .at[:CTX].set(ctx >> 1).reshape(CTX_ROWS, 128))
    parf = (jnp.zeros((CTX_PAD, LANES), jnp.float32)
            .at[:CTX, :].set((ctx & 1).astype(jnp.float32)[:, None]))
    path2_pad = jnp.zeros((PATH_PAD,), jnp.int32).at[:PATH].set(pidx >> 1)
    ctx_table2 = context_table.reshape(-1, W)
    node_table2 = node_table.reshape(-1, W)
    h, nrows = _sc_gather_pool(ctx2_pad, path2_pad, parf, ctx_table2,
                               node_table2)
    par_col = (jnp.zeros((PATH_PAD, 1), jnp.float32)
               .at[:PATH, 0].set((pidx & 1).astype(jnp.float32)))
    bits_col = (jnp.zeros((PATH_PAD, 1), jnp.float32)
                .at[:PATH, 0].set(code_bits.astype(jnp.float32)))
    mask_col = jnp.zeros((PATH_PAD, 1), jnp.float32).at[:PATH, 0].set(1.0)
    out = _loss_call(h.reshape(1, EMBED), nrows, par_col, bits_col, mask_col)
    return out[0, 0]


# speedup vs baseline: 1.5777x; 1.5777x over previous
import jax
import jax.numpy as jnp
from jax import lax
from jax.experimental import pallas as pl
from jax.experimental.pallas import tpu as pltpu

CTX = 200
PATH = 20
EMBED = 64
PATH_PAD = 32


def _body(ctx_idx_ref, path_idx_ref, bits_ref, ctx_table_ref, node_table_ref,
          o_ref, crows, nrows, sem):
    def issue_ctx(i, _):
        pltpu.make_async_copy(
            ctx_table_ref.at[pl.ds(ctx_idx_ref[i], 1)],
            crows.at[pl.ds(i, 1)], sem).start()
        return 0

    def issue_node(i, _):
        pltpu.make_async_copy(
            node_table_ref.at[pl.ds(path_idx_ref[i], 1)],
            nrows.at[pl.ds(i, 1)], sem).start()
        return 0

    lax.fori_loop(0, CTX, issue_ctx, 0)
    lax.fori_loop(0, PATH_PAD, issue_node, 0)

    def drain_ctx(i, _):
        pltpu.make_async_copy(
            ctx_table_ref.at[pl.ds(0, 1)], crows.at[pl.ds(i, 1)], sem).wait()
        return 0

    def drain_node(i, _):
        pltpu.make_async_copy(
            node_table_ref.at[pl.ds(0, 1)], nrows.at[pl.ds(i, 1)], sem).wait()
        return 0

    lax.fori_loop(0, CTX, drain_ctx, 0)
    lax.fori_loop(0, PATH_PAD, drain_node, 0)

    h = jnp.sum(crows[...], axis=0, keepdims=True) * (1.0 / CTX)
    n = nrows[...]
    b = bits_ref[...]
    t = jnp.sum(n * h, axis=1, keepdims=True)
    s = jax.nn.sigmoid(t)
    eps = 1e-9
    per = -b * jnp.log(s + eps) - (1.0 - b) * jnp.log(1.0 - s + eps)
    row = lax.broadcasted_iota(jnp.int32, (PATH_PAD, 1), 0)
    per = jnp.where(row < PATH, per, 0.0)
    o_ref[0, 0] = jnp.sum(per)


_call = pl.pallas_call(
    _body,
    in_specs=[
        pl.BlockSpec(memory_space=pltpu.SMEM),
        pl.BlockSpec(memory_space=pltpu.SMEM),
        pl.BlockSpec(memory_space=pltpu.VMEM),
        pl.BlockSpec(memory_space=pl.ANY),
        pl.BlockSpec(memory_space=pl.ANY),
    ],
    out_specs=pl.BlockSpec(memory_space=pltpu.SMEM),
    out_shape=jax.ShapeDtypeStruct((1, 1), jnp.float32),
    scratch_shapes=[
        pltpu.VMEM((CTX, EMBED), jnp.float32),
        pltpu.VMEM((PATH_PAD, EMBED), jnp.float32),
        pltpu.SemaphoreType.DMA,
    ],
)


def kernel(context_idx, path_indices, code_bits, context_table, node_table):
    ctx = jnp.asarray(context_idx, jnp.int32)
    pidx = jnp.asarray(path_indices, jnp.int32)
    path_pad = jnp.zeros((PATH_PAD,), jnp.int32).at[:PATH].set(pidx)
    bits_col = (jnp.zeros((PATH_PAD, 1), jnp.float32)
                .at[:PATH, 0].set(code_bits.astype(jnp.float32)))
    out = _call(ctx, path_pad, bits_col, context_table, node_table)
    return out[0, 0]
